# baseline (speedup 1.0000x reference)
import functools

import jax
import jax.numpy as jnp
from jax import lax
from jax.experimental import pallas as pl
from jax.experimental.pallas import tpu as pltpu
from jax.experimental.pallas import tpu_sc as plsc

NC = 2
NS = 16
W = NC * NS
L = 16
C = 40
G = 8
NBUF = 8


def _zero16():
    return jnp.zeros((L,), jnp.float32)


def _mesh():
    return plsc.VectorSubcoreMesh(
        core_axis_name="c", subcore_axis_name="s",
        num_cores=NC, num_subcores=NS)


def _make_deg(PN, EP):
    NCHW = EP // (C * W)
    NG = NCHW // G
    NWR = PN // NS

    @functools.partial(
        pl.kernel,
        out_type=(jax.ShapeDtypeStruct((PN,), jnp.float32),
                  jax.ShapeDtypeStruct((PN,), jnp.float32)),
        mesh=_mesh(),
        compiler_params=pltpu.CompilerParams(needs_layout_passes=False),
        scratch_types=[
            pltpu.VMEM_SHARED((PN,), jnp.float32),
            pltpu.VMEM((G, C), jnp.int32),
            pltpu.VMEM((C,), jnp.float32),
            pltpu.VMEM((NWR,), jnp.float32),
        ],
    )
    def deg_kernel(col_hbm, deg0_hbm, deg1_hbm, deg_sp, cidx, ones_v, zbuf):
        cid = lax.axis_index("c")
        sid = lax.axis_index("s")
        w = cid * NS + sid
        ebase = w * NCHW

        for i in range(C // L):
            ones_v[pl.ds(i * L, L)] = _zero16() + jnp.float32(1.0)

        def zb(i, _):
            zbuf[pl.ds(i * L, L)] = _zero16()
            return 0
        lax.fori_loop(0, NWR // L, zb, 0)
        pltpu.sync_copy(zbuf, deg_sp.at[pl.ds(sid * NWR, NWR)])
        plsc.subcore_barrier()

        def deg_body(g, _):
            pltpu.sync_copy(col_hbm.at[pl.ds(ebase + g * G, G), :], cidx)
            for b in range(G):
                pltpu.sync_copy(ones_v, deg_sp.at[cidx.at[b]], add=True)
            return 0
        lax.fori_loop(0, NG, deg_body, 0)
        plsc.subcore_barrier()

        @pl.when(cid == 0)
        def _():
            pltpu.sync_copy(deg_sp.at[pl.ds(sid * NWR, NWR)],
                            deg0_hbm.at[pl.ds(sid * NWR, NWR)])

        @pl.when(cid == 1)
        def _():
            pltpu.sync_copy(deg_sp.at[pl.ds(sid * NWR, NWR)],
                            deg1_hbm.at[pl.ds(sid * NWR, NWR)])

    return deg_kernel


def _make_edge(PN, D, EP):
    NCHW = EP // (C * W)
    NG = NCHW // G
    NWR = PN // NS
    DN = D // L

    @functools.partial(
        pl.kernel,
        out_type=(jax.ShapeDtypeStruct((PN, D), jnp.float32),
                  jax.ShapeDtypeStruct((PN, D), jnp.float32)),
        mesh=_mesh(),
        compiler_params=pltpu.CompilerParams(needs_layout_passes=False),
        scratch_types=[
            pltpu.VMEM_SHARED((PN, D), jnp.float32),
            pltpu.VMEM((2, G, C), jnp.int32),
            pltpu.VMEM((2, G, C), jnp.int32),
        ] + [pltpu.VMEM((C, D), jnp.float32) for _ in range(NBUF)]
          + [pltpu.SemaphoreType.DMA for _ in range(NBUF)]
          + [pltpu.SemaphoreType.DMA,
             pltpu.SemaphoreType.DMA],
    )
    def edge_kernel(t_hbm, row_hbm, col_hbm, u0_hbm, u1_hbm,
                    u_sp, ridx, cidx, *rest):
        bufs = rest[:NBUF]
        sems = rest[NBUF:2 * NBUF]
        ssem = rest[2 * NBUF]
        isem = rest[2 * NBUF + 1]
        cid = lax.axis_index("c")
        sid = lax.axis_index("s")
        w = cid * NS + sid
        ebase = w * NCHW

        def zero_buf(i, _):
            for c in range(DN):
                bufs[0][i, pl.ds(c * L, L)] = _zero16()
            return 0
        lax.fori_loop(0, C, zero_buf, 0)
        for k in range(NWR // C):
            pltpu.sync_copy(
                bufs[0], u_sp.at[pl.ds(sid * NWR + k * C, C), :])
        plsc.subcore_barrier()

        pltpu.async_copy(row_hbm.at[pl.ds(ebase, G), :], ridx.at[0], isem)
        pltpu.async_copy(col_hbm.at[pl.ds(ebase, G), :], cidx.at[0], isem)

        def gpair(g2, _):
            for p in (0, 1):
                g = 2 * g2 + p
                base = ebase + g * G
                pltpu.make_async_copy(
                    row_hbm.at[pl.ds(base, G), :], ridx.at[p], isem).wait()
                pltpu.make_async_copy(
                    col_hbm.at[pl.ds(base, G), :], cidx.at[p], isem).wait()

                @pl.when(g + 1 < NG)
                def _():
                    nb = ebase + (g + 1) * G
                    pltpu.async_copy(row_hbm.at[pl.ds(nb, G), :],
                                     ridx.at[1 - p], isem)
                    pltpu.async_copy(col_hbm.at[pl.ds(nb, G), :],
                                     cidx.at[1 - p], isem)

                pend = [
                    pltpu.async_copy(t_hbm.at[ridx.at[p, b]],
                                     bufs[b], sems[b])
                    for b in range(G)]
                spend = []
                for b in range(G):
                    pend[b].wait()
                    spend.append(pltpu.async_copy(
                        bufs[b], u_sp.at[cidx.at[p, b]], ssem, add=True))
                for d in spend:
                    d.wait()
            return 0
        lax.fori_loop(0, NG // 2, gpair, 0)
        plsc.subcore_barrier()

        @pl.when(cid == 0)
        def _():
            pltpu.sync_copy(u_sp.at[pl.ds(sid * NWR, NWR), :],
                            u0_hbm.at[pl.ds(sid * NWR, NWR), :])

        @pl.when(cid == 1)
        def _():
            pltpu.sync_copy(u_sp.at[pl.ds(sid * NWR, NWR), :],
                            u1_hbm.at[pl.ds(sid * NWR, NWR), :])

    return edge_kernel


def _prep_body(x_ref, d0_ref, d1_ref, t_ref, dis_ref, dis2_ref):
    deg = d0_ref[...] + d1_ref[...] + jnp.float32(1.0)
    dis = lax.rsqrt(deg)
    dis_ref[...] = dis
    dis2_ref[...] = jnp.float32(1.0) / deg
    t_ref[...] = x_ref[...] * dis


def _upd_body(u0_ref, u1_ref, t_ref, dis2_ref, o_ref):
    o_ref[...] = (u0_ref[...] + u1_ref[...] + t_ref[...]) * dis2_ref[...]


def _fin_body(u0_ref, u1_ref, t_ref, dis_ref, o_ref):
    v = (u0_ref[...] + u1_ref[...] + t_ref[...]) * dis_ref[...]
    o_ref[...] = jnp.sqrt(jnp.sum(v * v)).reshape(1, 1)


@functools.lru_cache(maxsize=None)
def _get_kernels(PN, D, EP):
    deg_k = _make_deg(PN, EP)
    edge_k = _make_edge(PN, D, EP)
    fvec = jax.ShapeDtypeStruct((PN, D), jnp.float32)
    fcol = jax.ShapeDtypeStruct((PN, 1), jnp.float32)
    prep = pl.pallas_call(
        _prep_body, out_shape=(fvec, fcol, fcol))
    upd = pl.pallas_call(_upd_body, out_shape=fvec)
    fin = pl.pallas_call(
        _fin_body, out_shape=jax.ShapeDtypeStruct((1, 1), jnp.float32))
    return deg_k, edge_k, prep, upd, fin


def kernel(x, adj, pool):
    N, D = x.shape
    E = adj.shape[1]
    PN = ((N + NS * 128 - 1) // (NS * 128)) * (NS * 128)
    EP = ((E + W * G * C * 2 - 1) // (W * G * C * 2)) * (W * G * C * 2)
    xp = jnp.pad(x, ((0, PN - N), (0, 0)))
    spare = max(PN - N, 1)
    padi = (N + jnp.arange(EP - E, dtype=jnp.int32) % spare).astype(jnp.int32)
    rowp = jnp.concatenate([adj[0], padi]).reshape(EP // C, C)
    colp = jnp.concatenate([adj[1], padi]).reshape(EP // C, C)

    deg_k, edge_k, prep, upd, fin = _get_kernels(PN, D, EP)
    deg0, deg1 = deg_k(colp)
    t, dis, dis2 = prep(xp, deg0.reshape(PN, 1), deg1.reshape(PN, 1))
    for r in range(3):
        u0, u1 = edge_k(t, rowp, colp)
        if r < 2:
            t = upd(u0, u1, t, dis2)
    out = fin(u0, u1, t, dis)
    return out[0, 0]


# speedup vs baseline: 1.1288x; 1.1288x over previous
import functools

import jax
import jax.numpy as jnp
from jax import lax
from jax.experimental import pallas as pl
from jax.experimental.pallas import tpu as pltpu
from jax.experimental.pallas import tpu_sc as plsc

NC = 2
NS = 16
W = NC * NS
L = 16
C = 128
G = 8
NBUF = 2


def _zero16():
    return jnp.zeros((L,), jnp.float32)


def _mesh():
    return plsc.VectorSubcoreMesh(
        core_axis_name="c", subcore_axis_name="s",
        num_cores=NC, num_subcores=NS)


def _make_deg(PN, EP):
    NCHW = EP // (C * W)
    NG = NCHW // G
    NWR = PN // NS

    @functools.partial(
        pl.kernel,
        out_type=(jax.ShapeDtypeStruct((PN,), jnp.float32),
                  jax.ShapeDtypeStruct((PN,), jnp.float32)),
        mesh=_mesh(),
        compiler_params=pltpu.CompilerParams(needs_layout_passes=False),
        scratch_types=[
            pltpu.VMEM_SHARED((PN,), jnp.float32),
            pltpu.VMEM((G, C), jnp.int32),
            pltpu.VMEM((C,), jnp.float32),
            pltpu.VMEM((NWR,), jnp.float32),
        ],
    )
    def deg_kernel(col_hbm, deg0_hbm, deg1_hbm, deg_sp, cidx, ones_v, zbuf):
        cid = lax.axis_index("c")
        sid = lax.axis_index("s")
        w = cid * NS + sid
        ebase = w * NCHW

        for i in range(C // L):
            ones_v[pl.ds(i * L, L)] = _zero16() + jnp.float32(1.0)

        def zb(i, _):
            zbuf[pl.ds(i * L, L)] = _zero16()
            return 0
        lax.fori_loop(0, NWR // L, zb, 0)
        pltpu.sync_copy(zbuf, deg_sp.at[pl.ds(sid * NWR, NWR)])
        plsc.subcore_barrier()

        def deg_body(g, _):
            pltpu.sync_copy(col_hbm.at[pl.ds(ebase + g * G, G), :], cidx)
            for b in range(G):
                pltpu.sync_copy(ones_v, deg_sp.at[cidx.at[b]], add=True)
            return 0
        lax.fori_loop(0, NG, deg_body, 0)
        plsc.subcore_barrier()

        @pl.when(cid == 0)
        def _():
            pltpu.sync_copy(deg_sp.at[pl.ds(sid * NWR, NWR)],
                            deg0_hbm.at[pl.ds(sid * NWR, NWR)])

        @pl.when(cid == 1)
        def _():
            pltpu.sync_copy(deg_sp.at[pl.ds(sid * NWR, NWR)],
                            deg1_hbm.at[pl.ds(sid * NWR, NWR)])

    return deg_kernel


def _make_edge(PN, D, EP):
    NCHW = EP // (C * W)
    NG = NCHW // G
    NWR = PN // NS
    DN = D // L

    @functools.partial(
        pl.kernel,
        out_type=(jax.ShapeDtypeStruct((PN, D), jnp.float32),
                  jax.ShapeDtypeStruct((PN, D), jnp.float32)),
        mesh=_mesh(),
        compiler_params=pltpu.CompilerParams(needs_layout_passes=False),
        scratch_types=[
            pltpu.VMEM_SHARED((PN, D), jnp.float32),
            pltpu.VMEM((2, G, C), jnp.int32),
            pltpu.VMEM((2, G, C), jnp.int32),
        ] + [pltpu.VMEM((C, D), jnp.float32) for _ in range(NBUF)]
          + [pltpu.SemaphoreType.DMA for _ in range(NBUF)]
          + [pltpu.SemaphoreType.DMA],
    )
    def edge_kernel(t_hbm, row_hbm, col_hbm, u0_hbm, u1_hbm,
                    u_sp, ridx, cidx, *rest):
        bufs = rest[:NBUF]
        sems = rest[NBUF:2 * NBUF]
        isem = rest[2 * NBUF]
        cid = lax.axis_index("c")
        sid = lax.axis_index("s")
        w = cid * NS + sid
        ebase = w * NCHW

        def zero_buf(i, _):
            for c in range(DN):
                bufs[0][i, pl.ds(c * L, L)] = _zero16()
            return 0
        lax.fori_loop(0, C, zero_buf, 0)
        for k in range(NWR // C):
            pltpu.sync_copy(
                bufs[0], u_sp.at[pl.ds(sid * NWR + k * C, C), :])
        plsc.subcore_barrier()

        pltpu.async_copy(row_hbm.at[pl.ds(ebase, G), :], ridx.at[0], isem)
        pltpu.async_copy(col_hbm.at[pl.ds(ebase, G), :], cidx.at[0], isem)

        def gpair(g2, _):
            for p in (0, 1):
                g = 2 * g2 + p
                base = ebase + g * G
                pltpu.make_async_copy(
                    row_hbm.at[pl.ds(base, G), :], ridx.at[p], isem).wait()
                pltpu.make_async_copy(
                    col_hbm.at[pl.ds(base, G), :], cidx.at[p], isem).wait()

                @pl.when(g + 1 < NG)
                def _():
                    nb = ebase + (g + 1) * G
                    pltpu.async_copy(row_hbm.at[pl.ds(nb, G), :],
                                     ridx.at[1 - p], isem)
                    pltpu.async_copy(col_hbm.at[pl.ds(nb, G), :],
                                     cidx.at[1 - p], isem)

                pend = [
                    pltpu.async_copy(t_hbm.at[ridx.at[p, b]],
                                     bufs[b], sems[b])
                    for b in range(NBUF)]
                for b in range(G):
                    pend[b].wait()
                    pltpu.sync_copy(bufs[b % NBUF],
                                    u_sp.at[cidx.at[p, b]], add=True)
                    if b + NBUF < G:
                        pend.append(pltpu.async_copy(
                            t_hbm.at[ridx.at[p, b + NBUF]],
                            bufs[b % NBUF], sems[b % NBUF]))
            return 0
        lax.fori_loop(0, NG // 2, gpair, 0)
        plsc.subcore_barrier()

        @pl.when(cid == 0)
        def _():
            pltpu.sync_copy(u_sp.at[pl.ds(sid * NWR, NWR), :],
                            u0_hbm.at[pl.ds(sid * NWR, NWR), :])

        @pl.when(cid == 1)
        def _():
            pltpu.sync_copy(u_sp.at[pl.ds(sid * NWR, NWR), :],
                            u1_hbm.at[pl.ds(sid * NWR, NWR), :])

    return edge_kernel


def _prep_body(x_ref, d0_ref, d1_ref, t_ref, dis_ref, dis2_ref):
    deg = d0_ref[...] + d1_ref[...] + jnp.float32(1.0)
    dis = lax.rsqrt(deg)
    dis_ref[...] = dis
    dis2_ref[...] = jnp.float32(1.0) / deg
    t_ref[...] = x_ref[...] * dis


def _upd_body(u0_ref, u1_ref, t_ref, dis2_ref, o_ref):
    o_ref[...] = (u0_ref[...] + u1_ref[...] + t_ref[...]) * dis2_ref[...]


def _fin_body(u0_ref, u1_ref, t_ref, dis_ref, o_ref):
    v = (u0_ref[...] + u1_ref[...] + t_ref[...]) * dis_ref[...]
    o_ref[...] = jnp.sqrt(jnp.sum(v * v)).reshape(1, 1)


@functools.lru_cache(maxsize=None)
def _get_kernels(PN, D, EP):
    deg_k = _make_deg(PN, EP)
    edge_k = _make_edge(PN, D, EP)
    fvec = jax.ShapeDtypeStruct((PN, D), jnp.float32)
    fcol = jax.ShapeDtypeStruct((PN, 1), jnp.float32)
    prep = pl.pallas_call(
        _prep_body, out_shape=(fvec, fcol, fcol))
    upd = pl.pallas_call(_upd_body, out_shape=fvec)
    fin = pl.pallas_call(
        _fin_body, out_shape=jax.ShapeDtypeStruct((1, 1), jnp.float32))
    return deg_k, edge_k, prep, upd, fin


def kernel(x, adj, pool):
    N, D = x.shape
    E = adj.shape[1]
    PN = ((N + NS * 128 - 1) // (NS * 128)) * (NS * 128)
    EP = ((E + W * G * C * 2 - 1) // (W * G * C * 2)) * (W * G * C * 2)
    xp = jnp.pad(x, ((0, PN - N), (0, 0)))
    spare = max(PN - N, 1)
    padi = (N + jnp.arange(EP - E, dtype=jnp.int32) % spare).astype(jnp.int32)
    rowp = jnp.concatenate([adj[0], padi]).reshape(EP // C, C)
    colp = jnp.concatenate([adj[1], padi]).reshape(EP // C, C)

    deg_k, edge_k, prep, upd, fin = _get_kernels(PN, D, EP)
    deg0, deg1 = deg_k(colp)
    t, dis, dis2 = prep(xp, deg0.reshape(PN, 1), deg1.reshape(PN, 1))
    for r in range(3):
        u0, u1 = edge_k(t, rowp, colp)
        if r < 2:
            t = upd(u0, u1, t, dis2)
    out = fin(u0, u1, t, dis)
    return out[0, 0]
